# Initial kernel scaffold; baseline (speedup 1.0000x reference)
import jax
import jax.numpy as jnp
from jax.experimental import pallas as pl


def kernel(x, edge_index, batch, W_pre1, b_pre1, gamma_pre, beta_pre, W_pre2, b_pre2, W_post1, b_post1, W_post2, b_post2, Wm1, bm1, Wm2, bm2, Wm3, bm3):
    raise NotImplementedError("write your pallas kernel here")


# speedup vs baseline: 2.6617x; 2.6617x over previous
import functools

import jax
import jax.numpy as jnp
from jax import lax
from jax.experimental import pallas as pl
from jax.experimental.pallas import tpu as pltpu
from jax.experimental.pallas import tpu_sc as plsc

N = 10000
F = 128
G = 64
OUT = 10

NC = 2
NS = 16
NW = NC * NS

C = 128
CH = 80
EW = C * CH
EPAD = EW * NW
SUB = 632
ACC_R = NS * SUB


def _sc_segsum_body(table_hbm, src_hbm, dst_hbm, zeros_hbm, out_hbm,
                    src_v, dst_v, rows_v, acc, sem):
    cid = lax.axis_index("c")
    sid = lax.axis_index("s")
    wid = sid * NC + cid

    pltpu.sync_copy(src_hbm.at[wid], src_v)
    pltpu.sync_copy(dst_hbm.at[wid], dst_v)

    pltpu.sync_copy(zeros_hbm, acc.at[pl.ds(sid * SUB, SUB)])
    plsc.subcore_barrier()

    def chunk(ci, carry):
        pltpu.async_copy(table_hbm.at[src_v.at[ci]], rows_v, sem).wait()
        pltpu.sync_copy(rows_v, acc.at[dst_v.at[ci]], add=True)
        return carry

    lax.fori_loop(0, CH, chunk, 0)
    plsc.subcore_barrier()

    pltpu.sync_copy(acc.at[pl.ds(sid * SUB, SUB)],
                    out_hbm.at[cid, pl.ds(sid * SUB, SUB)])


@functools.cache
def _get_sc_segsum():
    return pl.kernel(
        _sc_segsum_body,
        out_type=jax.ShapeDtypeStruct((NC, ACC_R, F), jnp.float32),
        mesh=plsc.VectorSubcoreMesh(core_axis_name="c", subcore_axis_name="s",
                                    num_cores=NC, num_subcores=NS),
        scratch_types=[
            pltpu.VMEM((CH, C), jnp.int32),
            pltpu.VMEM((CH, C), jnp.int32),
            pltpu.VMEM((C, F), jnp.float32),
            pltpu.VMEM_SHARED((ACC_R, F), jnp.float32),
            pltpu.SemaphoreType.DMA,
        ],
    )


def _elu(t):
    return jnp.where(t > 0, t, jnp.exp(t) - 1.0)


def _tc_pre_body(x_ref, parts_ref, w1_ref, b1_ref, g_ref, be_ref,
                 w2_ref, b2_ref, h_ref):
    z = x_ref[...] + parts_ref[0, :N, :] + parts_ref[1, :N, :]
    t = jnp.dot(z, w1_ref[...], preferred_element_type=jnp.float32) + b1_ref[...]
    mu = jnp.mean(t, axis=0, keepdims=True)
    var = jnp.mean((t - mu) * (t - mu), axis=0, keepdims=True)
    t = (t - mu) / jnp.sqrt(var + 1e-5) * g_ref[...] + be_ref[...]
    t = _elu(t)
    h = jnp.dot(t, w2_ref[...], preferred_element_type=jnp.float32) + b2_ref[...]
    h_ref[...] = _elu(h)


_tc_pre = pl.pallas_call(
    _tc_pre_body,
    out_shape=jax.ShapeDtypeStruct((N, F), jnp.float32),
)


def _tc_post_body(h_ref, parts_ref, batch_ref, wp1_ref, bp1_ref, wp2_ref,
                  bp2_ref, wm1_ref, bm1_ref, wm2_ref, bm2_ref, wm3_ref,
                  bm3_ref, o_ref):
    u = h_ref[...] + parts_ref[0, :N, :] + parts_ref[1, :N, :]
    a = _elu(jnp.dot(u, wp1_ref[...], preferred_element_type=jnp.float32)
             + bp1_ref[...])
    h2 = _elu(jnp.dot(a, wp2_ref[...], preferred_element_type=jnp.float32)
              + bp2_ref[...])
    gids = lax.broadcasted_iota(jnp.int32, (G, N), 0)
    onehot = (gids == batch_ref[...]).astype(jnp.float32)
    pooled = jnp.dot(onehot, h2, preferred_element_type=jnp.float32)
    o = _elu(jnp.dot(pooled, wm1_ref[...], preferred_element_type=jnp.float32)
             + bm1_ref[...])
    o = _elu(jnp.dot(o, wm2_ref[...], preferred_element_type=jnp.float32)
             + bm2_ref[...])
    o = jnp.dot(o, wm3_ref[...], preferred_element_type=jnp.float32) + bm3_ref[...]
    m = jnp.max(o, axis=-1, keepdims=True)
    lse = m + jnp.log(jnp.sum(jnp.exp(o - m), axis=-1, keepdims=True))
    o_ref[...] = o - lse


_tc_post = pl.pallas_call(
    _tc_post_body,
    out_shape=jax.ShapeDtypeStruct((G, OUT), jnp.float32),
)


def kernel(x, edge_index, batch, W_pre1, b_pre1, gamma_pre, beta_pre,
           W_pre2, b_pre2, W_post1, b_post1, W_post2, b_post2,
           Wm1, bm1, Wm2, bm2, Wm3, bm3):
    E = edge_index.shape[1]
    pad = EPAD - E
    src = jnp.concatenate([edge_index[0], jnp.zeros((pad,), jnp.int32)])
    dst = jnp.concatenate([edge_index[1], jnp.full((pad,), N, jnp.int32)])
    src3 = src.reshape(NW, CH, C)
    dst3 = dst.reshape(NW, CH, C)
    zeros_sub = jnp.zeros((SUB, F), jnp.float32)

    sc_segsum = _get_sc_segsum()
    parts1 = sc_segsum(x, src3, dst3, zeros_sub)
    h = _tc_pre(x, parts1, W_pre1, b_pre1.reshape(1, F),
                gamma_pre.reshape(1, F), beta_pre.reshape(1, F),
                W_pre2, b_pre2.reshape(1, F))
    parts2 = sc_segsum(h, src3, dst3, zeros_sub)
    logp = _tc_post(h, parts2, batch.reshape(1, N),
                    W_post1, b_post1.reshape(1, F),
                    W_post2, b_post2.reshape(1, F),
                    Wm1, bm1.reshape(1, F),
                    Wm2, bm2.reshape(1, F // 2),
                    Wm3, bm3.reshape(1, OUT))
    return (logp, jnp.float32(0.0))


# speedup vs baseline: 2.8905x; 1.0860x over previous
import functools

import jax
import jax.numpy as jnp
from jax import lax
from jax.experimental import pallas as pl
from jax.experimental.pallas import tpu as pltpu
from jax.experimental.pallas import tpu_sc as plsc

N = 10000
F = 128
G = 64
OUT = 10

NC = 2
NS = 16
NW = NC * NS

C = 128
CH = 80
H2 = 2
CHH = CH // H2
EW = C * CH
EPAD = EW * NW
SUB = 632
ACC_R = NS * SUB


def _sc_segsum_body(table_hbm, src_hbm, dst_hbm, zeros_hbm, out_hbm,
                    src_v, dst_v, rows_a, rows_b, acc, sem_a, sem_b):
    cid = lax.axis_index("c")
    sid = lax.axis_index("s")
    wid = sid * NC + cid

    pltpu.sync_copy(zeros_hbm, acc.at[pl.ds(sid * SUB, SUB)])
    plsc.subcore_barrier()

    def half(h, carry):
        pltpu.sync_copy(src_hbm.at[wid, pl.ds(h * CHH, CHH)], src_v)
        pltpu.sync_copy(dst_hbm.at[wid, pl.ds(h * CHH, CHH)], dst_v)
        pltpu.async_copy(table_hbm.at[src_v.at[0]], rows_a, sem_a)

        def pair(k, carry2):
            c0 = 2 * k
            pltpu.async_copy(table_hbm.at[src_v.at[c0 + 1]], rows_b, sem_b)
            pltpu.make_async_copy(table_hbm.at[src_v.at[0]], rows_a,
                                  sem_a).wait()
            pltpu.sync_copy(rows_a, acc.at[dst_v.at[c0]], add=True)
            nxt = lax.rem(c0 + 2, CHH)
            pltpu.async_copy(table_hbm.at[src_v.at[nxt]], rows_a, sem_a)
            pltpu.make_async_copy(table_hbm.at[src_v.at[0]], rows_b,
                                  sem_b).wait()
            pltpu.sync_copy(rows_b, acc.at[dst_v.at[c0 + 1]], add=True)
            return carry2

        lax.fori_loop(0, CHH // 2, pair, 0)
        pltpu.make_async_copy(table_hbm.at[src_v.at[0]], rows_a, sem_a).wait()
        return carry

    lax.fori_loop(0, H2, half, 0)
    plsc.subcore_barrier()

    pltpu.sync_copy(acc.at[pl.ds(sid * SUB, SUB)],
                    out_hbm.at[cid, pl.ds(sid * SUB, SUB)])


@functools.cache
def _get_sc_segsum():
    return pl.kernel(
        _sc_segsum_body,
        out_type=jax.ShapeDtypeStruct((NC, ACC_R, F), jnp.float32),
        mesh=plsc.VectorSubcoreMesh(core_axis_name="c", subcore_axis_name="s",
                                    num_cores=NC, num_subcores=NS),
        scratch_types=[
            pltpu.VMEM((CHH, C), jnp.int32),
            pltpu.VMEM((CHH, C), jnp.int32),
            pltpu.VMEM((C, F), jnp.float32),
            pltpu.VMEM((C, F), jnp.float32),
            pltpu.VMEM_SHARED((ACC_R, F), jnp.float32),
            pltpu.SemaphoreType.DMA,
            pltpu.SemaphoreType.DMA,
        ],
    )


def _elu(t):
    return jnp.where(t > 0, t, jnp.exp(t) - 1.0)


def _tc_pre_body(x_ref, parts_ref, w1_ref, b1_ref, g_ref, be_ref,
                 w2_ref, b2_ref, h_ref):
    z = x_ref[...] + parts_ref[0, :N, :] + parts_ref[1, :N, :]
    t = jnp.dot(z, w1_ref[...], preferred_element_type=jnp.float32) + b1_ref[...]
    mu = jnp.mean(t, axis=0, keepdims=True)
    var = jnp.mean((t - mu) * (t - mu), axis=0, keepdims=True)
    t = (t - mu) / jnp.sqrt(var + 1e-5) * g_ref[...] + be_ref[...]
    t = _elu(t)
    h = jnp.dot(t, w2_ref[...], preferred_element_type=jnp.float32) + b2_ref[...]
    h_ref[...] = _elu(h)


_tc_pre = pl.pallas_call(
    _tc_pre_body,
    out_shape=jax.ShapeDtypeStruct((N, F), jnp.float32),
)


def _tc_post_body(h_ref, parts_ref, batch_ref, wp1_ref, bp1_ref, wp2_ref,
                  bp2_ref, wm1_ref, bm1_ref, wm2_ref, bm2_ref, wm3_ref,
                  bm3_ref, o_ref):
    u = h_ref[...] + parts_ref[0, :N, :] + parts_ref[1, :N, :]
    a = _elu(jnp.dot(u, wp1_ref[...], preferred_element_type=jnp.float32)
             + bp1_ref[...])
    h2 = _elu(jnp.dot(a, wp2_ref[...], preferred_element_type=jnp.float32)
              + bp2_ref[...])
    gids = lax.broadcasted_iota(jnp.int32, (G, N), 0)
    onehot = (gids == batch_ref[...]).astype(jnp.float32)
    pooled = jnp.dot(onehot, h2, preferred_element_type=jnp.float32)
    o = _elu(jnp.dot(pooled, wm1_ref[...], preferred_element_type=jnp.float32)
             + bm1_ref[...])
    o = _elu(jnp.dot(o, wm2_ref[...], preferred_element_type=jnp.float32)
             + bm2_ref[...])
    o = jnp.dot(o, wm3_ref[...], preferred_element_type=jnp.float32) + bm3_ref[...]
    m = jnp.max(o, axis=-1, keepdims=True)
    lse = m + jnp.log(jnp.sum(jnp.exp(o - m), axis=-1, keepdims=True))
    o_ref[...] = o - lse


_tc_post = pl.pallas_call(
    _tc_post_body,
    out_shape=jax.ShapeDtypeStruct((G, OUT), jnp.float32),
)


def kernel(x, edge_index, batch, W_pre1, b_pre1, gamma_pre, beta_pre,
           W_pre2, b_pre2, W_post1, b_post1, W_post2, b_post2,
           Wm1, bm1, Wm2, bm2, Wm3, bm3):
    E = edge_index.shape[1]
    pad = EPAD - E
    src = jnp.concatenate([edge_index[0], jnp.zeros((pad,), jnp.int32)])
    dst = jnp.concatenate([edge_index[1], jnp.full((pad,), N, jnp.int32)])
    src3 = src.reshape(NW, CH, C)
    dst3 = dst.reshape(NW, CH, C)
    zeros_sub = jnp.zeros((SUB, F), jnp.float32)

    sc_segsum = _get_sc_segsum()
    parts1 = sc_segsum(x, src3, dst3, zeros_sub)
    h = _tc_pre(x, parts1, W_pre1, b_pre1.reshape(1, F),
                gamma_pre.reshape(1, F), beta_pre.reshape(1, F),
                W_pre2, b_pre2.reshape(1, F))
    parts2 = sc_segsum(h, src3, dst3, zeros_sub)
    logp = _tc_post(h, parts2, batch.reshape(1, N),
                    W_post1, b_post1.reshape(1, F),
                    W_post2, b_post2.reshape(1, F),
                    Wm1, bm1.reshape(1, F),
                    Wm2, bm2.reshape(1, F // 2),
                    Wm3, bm3.reshape(1, OUT))
    return (logp, jnp.float32(0.0))
